# baseline (speedup 1.0000x reference)
import functools

import jax
import jax.numpy as jnp
from jax import lax
from jax.experimental import pallas as pl
from jax.experimental.pallas import tpu as pltpu
from jax.experimental.pallas import tpu_sc as plsc

NB_USER = 1000000
NB_ITEM = 1000000
F = 32
B = 16384
RPL = 4
LINE = 128

NC, NS, L = 2, 16, 16
NW = NC * NS
BPW = B // NW
CH = 128
NCHUNK = BPW // CH
NBUF = 2


def _mf_body(users_hbm, items_hbm, ue_hbm, ie_hbm, ub_hbm, ib_hbm, gb_hbm,
             out_hbm,
             uidx, iidx, uq, iq, ubuf, ibuf, ubias, ibias, gbv, ob,
             su, si, sb, sg):
    wid = lax.axis_index("s") * NC + lax.axis_index("c")
    base = wid * BPW

    pltpu.sync_copy(users_hbm.at[pl.ds(base, BPW)], uidx)
    pltpu.sync_copy(items_hbm.at[pl.ds(base, BPW)], iidx)

    cub = pltpu.async_copy(ub_hbm.at[uidx], ubias, sb)
    cib = pltpu.async_copy(ib_hbm.at[iidx], ibias, sb)
    cgb = pltpu.async_copy(gb_hbm, gbv.at[pl.ds(0, 1)], sg)

    def lines(g, carry):
        s = pl.ds(g * L, L)
        uq[s] = lax.shift_right_logical(uidx[s], 2)
        iq[s] = lax.shift_right_logical(iidx[s], 2)
        return carry

    lax.fori_loop(0, BPW // L, lines, 0)

    def fire(c, slot):
        s = pl.ds(c * CH, CH)
        cu = pltpu.async_copy(ue_hbm.at[uq.at[s]], ubuf.at[slot], su)
        ci = pltpu.async_copy(ie_hbm.at[iq.at[s]], ibuf.at[slot], si)
        return cu, ci

    pend = [fire(0, 0)]

    lane = lax.iota(jnp.int32, L)

    for c in range(NCHUNK):
        slot = c % NBUF
        if c + 1 < NCHUNK:
            pend.append(fire(c + 1, (c + 1) % NBUF))
        cu, ci = pend[c]
        cu.wait()
        ci.wait()

        def group(g, carry, c=c, slot=slot):
            s = pl.ds(c * CH + g * L, L)
            ui = uidx[s]
            ii = iidx[s]
            uo = (ui & 3) * F
            io = (ii & 3) * F
            acc = jnp.zeros((L,), jnp.float32)
            for f in range(F):
                acc = acc + (plsc.load_gather(ubuf.at[slot], [lane + g * L, uo + f])
                             * plsc.load_gather(ibuf.at[slot], [lane + g * L, io + f]))
            ob[s] = acc
            return carry

        lax.fori_loop(0, CH // L, group, 0, unroll=True)

    cub.wait()
    cib.wait()
    cgb.wait()
    gb = gbv[...][0]

    def biasadd(g, carry):
        s = pl.ds(g * L, L)
        ob[s] = ob[s] + ubias[s] + ibias[s] + gb
        return carry

    lax.fori_loop(0, BPW // L, biasadd, 0)
    pltpu.sync_copy(ob, out_hbm.at[pl.ds(base, BPW)])


@jax.jit
def _mf(users, items, user_embeddings, item_embeddings, user_biases,
        item_biases, global_bias):
    mesh = plsc.VectorSubcoreMesh(core_axis_name="c", subcore_axis_name="s")
    run = pl.kernel(
        _mf_body,
        out_type=jax.ShapeDtypeStruct((B,), jnp.float32),
        mesh=mesh,
        compiler_params=pltpu.CompilerParams(
            needs_layout_passes=False, use_tc_tiling_on_sc=True),
        scratch_types=[
            pltpu.VMEM((BPW,), jnp.int32),
            pltpu.VMEM((BPW,), jnp.int32),
            pltpu.VMEM((BPW,), jnp.int32),
            pltpu.VMEM((BPW,), jnp.int32),
            pltpu.VMEM((NBUF, CH, LINE), jnp.float32),
            pltpu.VMEM((NBUF, CH, LINE), jnp.float32),
            pltpu.VMEM((BPW,), jnp.float32),
            pltpu.VMEM((BPW,), jnp.float32),
            pltpu.VMEM((L,), jnp.float32),
            pltpu.VMEM((BPW,), jnp.float32),
            pltpu.SemaphoreType.DMA,
            pltpu.SemaphoreType.DMA,
            pltpu.SemaphoreType.DMA,
            pltpu.SemaphoreType.DMA,
        ],
    )
    out = run(users, items,
              user_embeddings.reshape(NB_USER // RPL, LINE),
              item_embeddings.reshape(NB_ITEM // RPL, LINE),
              user_biases.reshape(NB_USER), item_biases.reshape(NB_ITEM),
              global_bias)
    return out.reshape(B, 1)


def kernel(users, items, user_embeddings, item_embeddings, user_biases,
           item_biases, global_bias):
    return _mf(users.astype(jnp.int32), items.astype(jnp.int32),
               user_embeddings, item_embeddings, user_biases, item_biases,
               global_bias)
